# baseline (speedup 1.0000x reference)
import functools
import jax
import jax.numpy as jnp
from jax import lax
from jax.experimental import pallas as pl
from jax.experimental.pallas import tpu as pltpu
from jax.experimental.pallas import tpu_sc as plsc

_DIM = 64
_SCALE = 8.0

_NC = 2
_NS = 16
_NW = _NC * _NS
_CH = 128
_NB = 4
_K = 2


def _make_kernel(n_seq):
    assert n_seq % _NB == 0 and n_seq // _NB >= 3
    n_groups = n_seq // _NB
    mesh = plsc.VectorSubcoreMesh(core_axis_name="c", subcore_axis_name="s")

    @functools.partial(
        pl.kernel,
        mesh=mesh,
        out_type=jax.ShapeDtypeStruct(
            (n_seq, _DIM // 8, _NW, 8, _CH), jnp.float32),
        scratch_types=[
            pltpu.VMEM((n_seq, _CH), jnp.int32),
            pltpu.VMEM((_NB, _CH, _DIM), jnp.float32),
            pltpu.VMEM((_NB, _DIM // 8, 8, _CH), jnp.float32),
            [pltpu.SemaphoreType.DMA] * _NB,
            [pltpu.SemaphoreType.DMA] * _NB,
        ],
        compiler_params=pltpu.CompilerParams(
            use_tc_tiling_on_sc=False, needs_layout_passes=False),
    )
    def k(idx_hbm, table_hbm, out_hbm, idx_v, rows_v, ob_v, sg, sw):
        wid = lax.axis_index("s") * _NC + lax.axis_index("c")
        pltpu.sync_copy(idx_hbm.at[wid], idx_v)

        def gather(s, j):
            return pltpu.make_async_copy(
                table_hbm.at[idx_v.at[s]], rows_v.at[j], sg[j])

        def write(s, j):
            return pltpu.make_async_copy(
                ob_v.at[j], out_hbm.at[s, :, wid, :, :], sw[j])

        def transpose_scale(j):
            for cg in range(_CH // 16):
                rowidx = lax.iota(jnp.int32, 16) + 16 * cg
                sl = pl.ds(16 * cg, 16)

                def dbody(d, carry):
                    dv = jnp.broadcast_to(d, (16,)).astype(jnp.int32)
                    v = plsc.load_gather(rows_v.at[j], [rowidx, dv])
                    ob_v[j, d // 8, d % 8, sl] = v * _SCALE
                    return carry

                lax.fori_loop(0, _DIM, dbody, 0, unroll=8)

        def step(s, j, start_next=True, wait_write=True):
            gather(s, j).wait()
            if wait_write:
                write(s - _NB, j).wait()
            transpose_scale(j)
            write(s, j).start()
            if start_next:
                gather(s + _K, (j + _K) % _NB).start()

        for j in range(_K):
            gather(j, j).start()
        for j in range(_NB):
            step(j, j, start_next=True, wait_write=False)

        def body(g, carry):
            s0 = g * _NB
            for j in range(_NB):
                step(s0 + j, j)
            return carry
        lax.fori_loop(1, n_groups - 1, body, 0)

        s0 = (n_groups - 1) * _NB
        for j in range(_NB):
            step(s0 + j, j, start_next=(j + _K < _NB))

        for j in range(_NB):
            write(s0 + j, j).wait()

    return k


@jax.jit
def kernel(x, table):
    bsz, seq = x.shape
    idx = x.reshape(_NW, _CH, seq).transpose(0, 2, 1).astype(jnp.int32)
    out5 = _make_kernel(seq)(idx, table)
    return out5.transpose(2, 4, 0, 1, 3).reshape(bsz, seq, _DIM)
